# Initial kernel scaffold; baseline (speedup 1.0000x reference)
import jax
import jax.numpy as jnp
from jax.experimental import pallas as pl


def kernel(x, codebook):
    raise NotImplementedError("write your pallas kernel here")


# speedup vs baseline: 2.7002x; 2.7002x over previous
import functools

import jax
import jax.numpy as jnp
from jax.experimental import pallas as pl

_BB = 256
_KC = 128


def _vq_body(x_ref, c_ref, zq_ref, ids_ref):
    xb = x_ref[...]
    BB = xb.shape[0]
    K, D = c_ref.shape
    nkc = K // _KC

    best_d = jnp.full((1, BB), jnp.inf, jnp.float32)
    best_i = jnp.zeros((1, BB), jnp.int32)
    for j in range(nkc):
        cb = c_ref[pl.ds(j * _KC, _KC), :]
        scores = jax.lax.dot_general(
            cb, xb,
            dimension_numbers=(((1,), (1,)), ((), ())),
            preferred_element_type=jnp.float32,
        )
        cn = jnp.sum(cb * cb, axis=1, keepdims=True)
        dist = cn - 2.0 * scores
        m = jnp.min(dist, axis=0, keepdims=True)
        iota = jax.lax.broadcasted_iota(jnp.int32, dist.shape, 0)
        li = jnp.min(jnp.where(dist == m, iota, _KC), axis=0,
                     keepdims=True) + j * _KC
        better = m < best_d
        best_d = jnp.where(better, m, best_d)
        best_i = jnp.where(better, li, best_i)

    ids_ref[...] = best_i.reshape((BB,))

    bi_col = best_i.reshape((BB, 1))
    acc = jnp.zeros((BB, D), jnp.float32)
    for j in range(nkc):
        cb = c_ref[pl.ds(j * _KC, _KC), :]
        iota = jax.lax.broadcasted_iota(jnp.int32, (BB, _KC), 1) + j * _KC
        onehot = (iota == bi_col).astype(jnp.float32)
        acc = acc + jax.lax.dot_general(
            onehot, cb,
            dimension_numbers=(((1,), (0,)), ((), ())),
            preferred_element_type=jnp.float32,
        )
    zq_ref[...] = acc


@jax.jit
def kernel(x, codebook):
    B, D = x.shape
    K = codebook.shape[0]
    grid = (B // _BB,)
    zq, ids = pl.pallas_call(
        _vq_body,
        grid=grid,
        in_specs=[
            pl.BlockSpec((_BB, D), lambda i: (i, 0)),
            pl.BlockSpec((K, D), lambda i: (0, 0)),
        ],
        out_specs=[
            pl.BlockSpec((_BB, D), lambda i: (i, 0)),
            pl.BlockSpec((_BB,), lambda i: (i,)),
        ],
        out_shape=[
            jax.ShapeDtypeStruct((B, D), jnp.float32),
            jax.ShapeDtypeStruct((B,), jnp.int32),
        ],
    )(x, codebook)
    return (zq, ids)


# speedup vs baseline: 2.8099x; 1.0406x over previous
import functools

import jax
import jax.numpy as jnp
from jax import lax
from jax.experimental import pallas as pl
from jax.experimental.pallas import tpu as pltpu
from jax.experimental.pallas import tpu_sc as plsc

_BB = 256
_KC = 512


def _argmin_body(x_ref, c_ref, ids_ref):
    xb = x_ref[...]
    BB = xb.shape[0]
    K, D = c_ref.shape
    nkc = K // _KC

    best_d = jnp.full((1, BB), jnp.inf, jnp.float32)
    best_i = jnp.zeros((1, BB), jnp.int32)
    for j in range(nkc):
        cb = c_ref[pl.ds(j * _KC, _KC), :]
        scores = jax.lax.dot_general(
            cb, xb,
            dimension_numbers=(((1,), (1,)), ((), ())),
            preferred_element_type=jnp.float32,
        )
        cn = jnp.sum(cb * cb, axis=1, keepdims=True)
        dist = cn - 2.0 * scores
        m = jnp.min(dist, axis=0, keepdims=True)
        iota = jax.lax.broadcasted_iota(jnp.int32, dist.shape, 0)
        li = jnp.min(jnp.where(dist == m, iota, _KC), axis=0,
                     keepdims=True) + j * _KC
        better = m < best_d
        best_d = jnp.where(better, m, best_d)
        best_i = jnp.where(better, li, best_i)

    ids_ref[...] = best_i.reshape((BB,))


def _tc_argmin(x, codebook):
    B, D = x.shape
    K = codebook.shape[0]
    return pl.pallas_call(
        _argmin_body,
        grid=(B // _BB,),
        in_specs=[
            pl.BlockSpec((_BB, D), lambda i: (i, 0)),
            pl.BlockSpec((K, D), lambda i: (0, 0)),
        ],
        out_specs=pl.BlockSpec((_BB,), lambda i: (i,)),
        out_shape=jax.ShapeDtypeStruct((B,), jnp.int32),
    )(x, codebook)


def _make_sc_gather(B, D):
    info = plsc.get_sparse_core_info()
    nw = info.num_cores * info.num_subcores
    b_per_w = B // nw
    mesh = plsc.VectorSubcoreMesh(core_axis_name="c", subcore_axis_name="s")

    @functools.partial(
        pl.kernel, mesh=mesh,
        out_type=jax.ShapeDtypeStruct((B, D), jnp.float32),
        compiler_params=pltpu.CompilerParams(use_tc_tiling_on_sc=False),
        scratch_types=[
            pltpu.VMEM((b_per_w,), jnp.int32),
            pltpu.VMEM((b_per_w, D), jnp.float32),
            pltpu.SemaphoreType.DMA,
        ],
    )
    def gather_rows(table_hbm, idx_hbm, out_hbm, idx_v, rows_v, sem):
        wid = lax.axis_index("s") * info.num_cores + lax.axis_index("c")
        base = wid * b_per_w
        pltpu.sync_copy(idx_hbm.at[pl.ds(base, b_per_w)], idx_v)
        pltpu.async_copy(table_hbm.at[idx_v], rows_v, sem).wait()
        pltpu.sync_copy(rows_v, out_hbm.at[pl.ds(base, b_per_w)])

    return gather_rows


@jax.jit
def kernel(x, codebook):
    B, D = x.shape
    ids = _tc_argmin(x, codebook)
    zq = _make_sc_gather(B, D)(codebook, ids)
    return (zq, ids)


# speedup vs baseline: 4.0368x; 1.4366x over previous
import functools

import jax
import jax.numpy as jnp
from jax import lax
from jax.experimental import pallas as pl
from jax.experimental.pallas import tpu as pltpu
from jax.experimental.pallas import tpu_sc as plsc

_BB = 1024
_TB = 256
_KC = 512


def _argmin_body(x_ref, c_ref, ids_ref):
    K, D = c_ref.shape
    nkc = K // _KC

    for t in range(_BB // _TB):
        xb = x_ref[pl.ds(t * _TB, _TB), :]
        best_d = jnp.full((1, _TB), jnp.inf, jnp.float32)
        best_i = jnp.zeros((1, _TB), jnp.int32)
        for j in range(nkc):
            cb = c_ref[pl.ds(j * _KC, _KC), :]
            scores = jax.lax.dot_general(
                cb, xb,
                dimension_numbers=(((1,), (1,)), ((), ())),
                preferred_element_type=jnp.float32,
            )
            cn = jnp.sum(cb * cb, axis=1, keepdims=True)
            dist = cn - 2.0 * scores
            m = jnp.min(dist, axis=0, keepdims=True)
            iota = jax.lax.broadcasted_iota(jnp.int32, dist.shape, 0)
            li = jnp.min(jnp.where(dist == m, iota, _KC), axis=0,
                         keepdims=True) + j * _KC
            better = m < best_d
            best_d = jnp.where(better, m, best_d)
            best_i = jnp.where(better, li, best_i)

        ids_ref[pl.ds(t * _TB, _TB)] = best_i.reshape((_TB,))


def _tc_argmin(x, codebook):
    B, D = x.shape
    K = codebook.shape[0]
    return pl.pallas_call(
        _argmin_body,
        grid=(B // _BB,),
        in_specs=[
            pl.BlockSpec((_BB, D), lambda i: (i, 0)),
            pl.BlockSpec((K, D), lambda i: (0, 0)),
        ],
        out_specs=pl.BlockSpec((_BB,), lambda i: (i,)),
        out_shape=jax.ShapeDtypeStruct((B,), jnp.int32),
    )(x, codebook)


def _make_sc_gather(B, D):
    info = plsc.get_sparse_core_info()
    nw = info.num_cores * info.num_subcores
    b_per_w = B // nw
    mesh = plsc.VectorSubcoreMesh(core_axis_name="c", subcore_axis_name="s")

    @functools.partial(
        pl.kernel, mesh=mesh,
        out_type=jax.ShapeDtypeStruct((B, D), jnp.float32),
        compiler_params=pltpu.CompilerParams(use_tc_tiling_on_sc=False),
        scratch_types=[
            pltpu.VMEM((b_per_w,), jnp.int32),
            pltpu.VMEM((b_per_w, D), jnp.float32),
            pltpu.SemaphoreType.DMA,
        ],
    )
    def gather_rows(table_hbm, idx_hbm, out_hbm, idx_v, rows_v, sem):
        wid = lax.axis_index("s") * info.num_cores + lax.axis_index("c")
        base = wid * b_per_w
        pltpu.sync_copy(idx_hbm.at[pl.ds(base, b_per_w)], idx_v)
        pltpu.async_copy(table_hbm.at[idx_v], rows_v, sem).wait()
        pltpu.sync_copy(rows_v, out_hbm.at[pl.ds(base, b_per_w)])

    return gather_rows


@jax.jit
def kernel(x, codebook):
    B, D = x.shape
    ids = _tc_argmin(x, codebook)
    zq = _make_sc_gather(B, D)(codebook, ids)
    return (zq, ids)


# speedup vs baseline: 4.2670x; 1.0570x over previous
import functools

import jax
import jax.numpy as jnp
from jax import lax
from jax.experimental import pallas as pl
from jax.experimental.pallas import tpu as pltpu
from jax.experimental.pallas import tpu_sc as plsc

_BB = 2048
_TB = 256
_KC = 512


def _argmin_body(x_ref, c_ref, ids_ref):
    K, D = c_ref.shape
    nkc = K // _KC

    for t in range(_BB // _TB):
        xb = x_ref[pl.ds(t * _TB, _TB), :]
        best_d = jnp.full((1, _TB), jnp.inf, jnp.float32)
        best_i = jnp.zeros((1, _TB), jnp.float32)
        for j in range(nkc):
            cb = c_ref[pl.ds(j * _KC, _KC), :]
            scores = jax.lax.dot_general(
                cb, xb,
                dimension_numbers=(((1,), (1,)), ((), ())),
                preferred_element_type=jnp.float32,
            )
            cn = jnp.sum(cb * cb, axis=1, keepdims=True)
            dist = cn - 2.0 * scores
            m = jnp.min(dist, axis=0, keepdims=True)
            iota = jax.lax.broadcasted_iota(
                jnp.int32, dist.shape, 0).astype(jnp.float32)
            li = jnp.min(jnp.where(dist == m, iota, float(_KC)), axis=0,
                         keepdims=True) + float(j * _KC)
            better = m < best_d
            best_d = jnp.where(better, m, best_d)
            best_i = jnp.where(better, li, best_i)

        ids_ref[pl.ds(t * _TB, _TB)] = best_i.reshape((_TB,)).astype(jnp.int32)


def _tc_argmin(x, codebook):
    B, D = x.shape
    K = codebook.shape[0]
    return pl.pallas_call(
        _argmin_body,
        grid=(B // _BB,),
        in_specs=[
            pl.BlockSpec((_BB, D), lambda i: (i, 0)),
            pl.BlockSpec((K, D), lambda i: (0, 0)),
        ],
        out_specs=pl.BlockSpec((_BB,), lambda i: (i,)),
        out_shape=jax.ShapeDtypeStruct((B,), jnp.int32),
    )(x, codebook)


def _make_sc_gather(B, D):
    info = plsc.get_sparse_core_info()
    nw = info.num_cores * info.num_subcores
    b_per_w = B // nw
    mesh = plsc.VectorSubcoreMesh(core_axis_name="c", subcore_axis_name="s")

    @functools.partial(
        pl.kernel, mesh=mesh,
        out_type=jax.ShapeDtypeStruct((B, D), jnp.float32),
        compiler_params=pltpu.CompilerParams(use_tc_tiling_on_sc=False),
        scratch_types=[
            pltpu.VMEM((b_per_w,), jnp.int32),
            pltpu.VMEM((b_per_w, D), jnp.float32),
            pltpu.SemaphoreType.DMA,
        ],
    )
    def gather_rows(table_hbm, idx_hbm, out_hbm, idx_v, rows_v, sem):
        wid = lax.axis_index("s") * info.num_cores + lax.axis_index("c")
        base = wid * b_per_w
        pltpu.sync_copy(idx_hbm.at[pl.ds(base, b_per_w)], idx_v)
        pltpu.async_copy(table_hbm.at[idx_v], rows_v, sem).wait()
        pltpu.sync_copy(rows_v, out_hbm.at[pl.ds(base, b_per_w)])

    return gather_rows


@jax.jit
def kernel(x, codebook):
    B, D = x.shape
    ids = _tc_argmin(x, codebook)
    zq = _make_sc_gather(B, D)(codebook, ids)
    return (zq, ids)


# speedup vs baseline: 4.4975x; 1.0540x over previous
import functools

import jax
import jax.numpy as jnp
from jax import lax
from jax.experimental import pallas as pl
from jax.experimental.pallas import tpu as pltpu
from jax.experimental.pallas import tpu_sc as plsc

_BB = 2048
_TB = 256
_KC = 128


def _argmin_body(x_ref, c_ref, ids_ref):
    K, D = c_ref.shape
    nkc = K // _KC

    nt = _BB // _TB
    best_d = [jnp.full((1, _TB), jnp.inf, jnp.float32) for _ in range(nt)]
    best_i = [jnp.zeros((1, _TB), jnp.float32) for _ in range(nt)]
    for j in range(nkc):
        cb = c_ref[pl.ds(j * _KC, _KC), :]
        cbm = cb * -2.0
        cn = jnp.sum(cb * cb, axis=1, keepdims=True)
        for t in range(nt):
            xb = x_ref[pl.ds(t * _TB, _TB), :]
            scores = jax.lax.dot_general(
                cbm, xb,
                dimension_numbers=(((1,), (1,)), ((), ())),
                preferred_element_type=jnp.float32,
            )
            dist = cn + scores
            m = jnp.min(dist, axis=0, keepdims=True)
            iota = jax.lax.broadcasted_iota(
                jnp.int32, dist.shape, 0).astype(jnp.float32)
            li = jnp.min(jnp.where(dist == m, iota, float(_KC)), axis=0,
                         keepdims=True) + float(j * _KC)
            better = m < best_d[t]
            best_d[t] = jnp.where(better, m, best_d[t])
            best_i[t] = jnp.where(better, li, best_i[t])

    for t in range(nt):
        ids_ref[pl.ds(t * _TB, _TB)] = (
            best_i[t].reshape((_TB,)).astype(jnp.int32))


def _tc_argmin(x, codebook):
    B, D = x.shape
    K = codebook.shape[0]
    return pl.pallas_call(
        _argmin_body,
        grid=(B // _BB,),
        in_specs=[
            pl.BlockSpec((_BB, D), lambda i: (i, 0)),
            pl.BlockSpec((K, D), lambda i: (0, 0)),
        ],
        out_specs=pl.BlockSpec((_BB,), lambda i: (i,)),
        out_shape=jax.ShapeDtypeStruct((B,), jnp.int32),
    )(x, codebook)


def _make_sc_gather(B, D):
    info = plsc.get_sparse_core_info()
    nw = info.num_cores * info.num_subcores
    b_per_w = B // nw
    mesh = plsc.VectorSubcoreMesh(core_axis_name="c", subcore_axis_name="s")

    @functools.partial(
        pl.kernel, mesh=mesh,
        out_type=jax.ShapeDtypeStruct((B, D), jnp.float32),
        compiler_params=pltpu.CompilerParams(use_tc_tiling_on_sc=False),
        scratch_types=[
            pltpu.VMEM((b_per_w,), jnp.int32),
            pltpu.VMEM((b_per_w, D), jnp.float32),
            pltpu.SemaphoreType.DMA,
        ],
    )
    def gather_rows(table_hbm, idx_hbm, out_hbm, idx_v, rows_v, sem):
        wid = lax.axis_index("s") * info.num_cores + lax.axis_index("c")
        base = wid * b_per_w
        pltpu.sync_copy(idx_hbm.at[pl.ds(base, b_per_w)], idx_v)
        pltpu.async_copy(table_hbm.at[idx_v], rows_v, sem).wait()
        pltpu.sync_copy(rows_v, out_hbm.at[pl.ds(base, b_per_w)])

    return gather_rows


@jax.jit
def kernel(x, codebook):
    B, D = x.shape
    ids = _tc_argmin(x, codebook)
    zq = _make_sc_gather(B, D)(codebook, ids)
    return (zq, ids)
